# Initial kernel scaffold; baseline (speedup 1.0000x reference)
import jax
import jax.numpy as jnp
from jax.experimental import pallas as pl


def kernel(x, mask, table):
    raise NotImplementedError("write your pallas kernel here")


# speedup vs baseline: 2.4454x; 2.4454x over previous
import functools

import jax
import jax.numpy as jnp
from jax import lax
from jax.experimental import pallas as pl
from jax.experimental.pallas import tpu as pltpu
from jax.experimental.pallas import tpu_sc as plsc

B = 16384
L = 50
D = 64
LANES = 16

_info = plsc.get_sparse_core_info()
NC = _info.num_cores
NS = _info.num_subcores
NW = NC * NS
ROWS_PER_W = B // NW
C = 16
NCHUNK = ROWS_PER_W // C


MPAD = 64


def _body(x_hbm, mask_hbm, table_hbm, out_hbm, idx_v, mask_v, rows_v, out_v, gsem):
    wid = lax.axis_index("s") * NC + lax.axis_index("c")
    row0 = wid * ROWS_PER_W

    def chunk(ch, carry):
        base = row0 + ch * C
        off = pl.multiple_of(base * L, 8)
        pltpu.sync_copy(x_hbm.at[pl.ds(off, C * L)], idx_v)
        pltpu.sync_copy(mask_hbm.at[pl.ds(base, C), :], mask_v)
        pltpu.async_copy(table_hbm.at[idx_v], rows_v, gsem).wait()

        def row(b, inner):
            mv = [mask_v[b, pl.ds(k * LANES, LANES)] for k in range(MPAD // LANES)]
            accs = [jnp.zeros((LANES,), jnp.float32) for _ in range(D // LANES)]
            cnt = jnp.float32(0.0)
            for l in range(L):
                m = mv[l // LANES][l % LANES]
                cnt = cnt + m
                for d in range(D // LANES):
                    accs[d] = accs[d] + rows_v[b * L + l, pl.ds(d * LANES, LANES)] * m
            denom = jnp.maximum(cnt, jnp.float32(1.0))
            for d in range(D // LANES):
                out_v[b, pl.ds(d * LANES, LANES)] = accs[d] / denom
            return inner

        lax.fori_loop(0, C, row, 0)
        pltpu.sync_copy(out_v, out_hbm.at[pl.ds(base, C), :])
        return carry

    lax.fori_loop(0, NCHUNK, chunk, 0)


_kern = pl.kernel(
    _body,
    out_type=jax.ShapeDtypeStruct((B, D), jnp.float32),
    mesh=plsc.VectorSubcoreMesh(core_axis_name="c", subcore_axis_name="s"),
    compiler_params=pltpu.CompilerParams(use_tc_tiling_on_sc=False),
    scratch_types=[
        pltpu.VMEM((C * L,), jnp.int32),
        pltpu.VMEM((C, MPAD), jnp.float32),
        pltpu.VMEM((C * L, D), jnp.float32),
        pltpu.VMEM((C, D), jnp.float32),
        pltpu.SemaphoreType.DMA,
    ],
)


@jax.jit
def kernel(x, mask, table):
    mask_p = jnp.pad(mask, ((0, 0), (0, MPAD - L)))
    return _kern(x.reshape(-1), mask_p, table)
